# Initial kernel scaffold; baseline (speedup 1.0000x reference)
import jax
import jax.numpy as jnp
from jax.experimental import pallas as pl


def kernel(traj, emb_weight):
    raise NotImplementedError("write your pallas kernel here")


# speedup vs baseline: 1.6844x; 1.6844x over previous
import jax
import jax.numpy as jnp
from jax import lax
from jax.experimental import pallas as pl
from jax.experimental.pallas import tpu as pltpu
from jax.experimental.pallas import tpu_sc as plsc

BATCH = 16384
HIST_LEN = 50
D = 64
N = BATCH * HIST_LEN
NC, NS = 2, 16
NW = NC * NS
C = 128
CPW = N // (NW * C)


def _emb_body(idx_hbm, table_hbm, out_hbm, idx_v, rows_v, gsem):
    wid = lax.axis_index("s") * NC + lax.axis_index("c")
    row0 = wid * CPW
    pltpu.sync_copy(idx_hbm.at[pl.ds(row0, CPW)], idx_v)

    def body(j, carry):
        pltpu.async_copy(table_hbm.at[idx_v.at[j]], rows_v, gsem).wait()
        pltpu.sync_copy(rows_v, out_hbm.at[pl.ds((row0 + j) * C, C)])
        return carry

    lax.fori_loop(0, CPW, body, 0)


@jax.jit
def kernel(traj, emb_weight):
    idx = traj.reshape(N // C, C).astype(jnp.int32)
    out = pl.kernel(
        _emb_body,
        out_type=jax.ShapeDtypeStruct((N, D), jnp.float32),
        mesh=plsc.VectorSubcoreMesh(core_axis_name="c", subcore_axis_name="s"),
        compiler_params=pltpu.CompilerParams(use_tc_tiling_on_sc=False),
        scratch_types=[
            pltpu.VMEM((CPW, C), jnp.int32),
            pltpu.VMEM((C, D), jnp.float32),
            pltpu.SemaphoreType.DMA,
        ],
    )(idx, emb_weight)
    return out.reshape(BATCH, HIST_LEN, D)


# speedup vs baseline: 1.8757x; 1.1135x over previous
import jax
import jax.numpy as jnp
from jax import lax
from jax.experimental import pallas as pl
from jax.experimental.pallas import tpu as pltpu
from jax.experimental.pallas import tpu_sc as plsc

BATCH = 16384
HIST_LEN = 50
D = 64
N = BATCH * HIST_LEN
NC, NS = 2, 16
NW = NC * NS
C = 128
CPW = N // (NW * C)
NBUF = 8
NGRP = CPW // NBUF


def _emb_body(idx_hbm, table_hbm, out_hbm, idx_v, rows, gsem, wsem):
    wid = lax.axis_index("s") * NC + lax.axis_index("c")
    row0 = wid * CPW
    pltpu.sync_copy(idx_hbm.at[pl.ds(row0, CPW)], idx_v)

    def fire_gather(j, b):
        pltpu.async_copy(table_hbm.at[idx_v.at[j]], rows[b], gsem[b])

    def wait_gather(b):
        pltpu.make_async_copy(table_hbm.at[idx_v.at[0]], rows[b], gsem[b]).wait()

    def fire_write(j, b):
        pltpu.async_copy(rows[b], out_hbm.at[pl.ds((row0 + j) * C, C)], wsem[b])

    def wait_write(b):
        pltpu.make_async_copy(rows[b], out_hbm.at[pl.ds(row0 * C, C)], wsem[b]).wait()

    def step(j, b, do_wait_write, do_fire):
        wait_gather(b)
        fire_write(j, b)
        if do_fire:
            bf = (b + NBUF - 1) % NBUF
            if do_wait_write:
                wait_write(bf)
            fire_gather(j + NBUF - 1, bf)

    for b in range(NBUF - 1):
        fire_gather(b, b)
    for b in range(NBUF):
        step(b, b, do_wait_write=(b > 0), do_fire=True)

    def group(g, carry):
        for b in range(NBUF):
            step(g * NBUF + b, b, do_wait_write=True, do_fire=True)
        return carry

    lax.fori_loop(1, NGRP - 1, group, 0)

    j0 = (NGRP - 1) * NBUF
    for b in range(NBUF):
        step(j0 + b, b, do_wait_write=(b == 0), do_fire=(b == 0))

    for b in range(NBUF):
        wait_write(b)


@jax.jit
def kernel(traj, emb_weight):
    idx = traj.reshape(N // C, C).astype(jnp.int32)
    out = pl.kernel(
        _emb_body,
        out_type=jax.ShapeDtypeStruct((N, D), jnp.float32),
        mesh=plsc.VectorSubcoreMesh(core_axis_name="c", subcore_axis_name="s"),
        compiler_params=pltpu.CompilerParams(use_tc_tiling_on_sc=False),
        scratch_types=[
            pltpu.VMEM((CPW, C), jnp.int32),
            [pltpu.VMEM((C, D), jnp.float32) for _ in range(NBUF)],
            [pltpu.SemaphoreType.DMA for _ in range(NBUF)],
            [pltpu.SemaphoreType.DMA for _ in range(NBUF)],
        ],
    )(idx, emb_weight)
    return out.reshape(BATCH, HIST_LEN, D)
